# Initial kernel scaffold; baseline (speedup 1.0000x reference)
import jax
import jax.numpy as jnp
from jax.experimental import pallas as pl


def kernel(measurement_times, measurement_values, W, b):
    raise NotImplementedError("write your pallas kernel here")


# speedup vs baseline: 10.0944x; 10.0944x over previous
import jax
import jax.numpy as jnp
from jax.experimental import pallas as pl
from jax.experimental.pallas import tpu as pltpu

_N = 8192
_T = 64
_D = 128
_B = 16
_ROWS = 1024


def _body(t_ref, v_ref, w_ref, b_ref, o_ref):
    t = t_ref[...]
    v = v_ref[...]
    s = jnp.sum(t, axis=1, keepdims=True)
    c = jnp.sum(t * v, axis=1, keepdims=True)
    scale = jnp.where(s == 0.0, 0.0, c / jnp.where(s == 0.0, 1.0, s))
    zero = jnp.where(s == 0.0, 0.0, 1.0)
    o_ref[...] = scale * w_ref[...] + zero * b_ref[...]


def kernel(measurement_times, measurement_values, W, b):
    grid = (_N // _ROWS,)
    out = pl.pallas_call(
        _body,
        grid=grid,
        in_specs=[
            pl.BlockSpec((_ROWS, _T), lambda i: (i, 0)),
            pl.BlockSpec((_ROWS, _T), lambda i: (i, 0)),
            pl.BlockSpec((_ROWS, _D), lambda i: (i, 0)),
            pl.BlockSpec((_ROWS, _D), lambda i: (i, 0)),
        ],
        out_specs=pl.BlockSpec((_ROWS, _D), lambda i: (i, 0)),
        out_shape=jax.ShapeDtypeStruct((_N, _D), jnp.float32),
    )(measurement_times, measurement_values, W, b)
    return out.reshape(_B, _N // _B, _D)
